# baseline (speedup 1.0000x reference)
import functools

import jax
import jax.numpy as jnp
from jax import lax
from jax.experimental import pallas as pl
from jax.experimental.pallas import tpu as pltpu
from jax.experimental.pallas import tpu_sc as plsc

_N = 10000
_E = 320000
_D_IN = 128
_D_HID = 16
_D_OUT = 40

_NC = 2
_NS = 16
_NW = _NC * _NS
_CH = 128
_NCHK = 80
_EPW = _CH * _NCHK
_EPAD = _EPW * _NW
_RPS = _N // _NS
_NDIS = 128
_NPAD = _N + _NDIS
_NBUF = 4

_BLK = 1000
_NBLK = _N // _BLK

_SC_PARAMS = pltpu.CompilerParams(use_tc_tiling_on_sc=False)
_MESH = plsc.VectorSubcoreMesh(core_axis_name="c", subcore_axis_name="s")


def _prop_body(g_hbm, row_hbm, col_hbm, out_hbm,
               row_v, col_v,
               rows_a, rows_b, rows_c, rows_d,
               acc_sh,
               gsem_a, gsem_b, gsem_c, gsem_d):
    c = lax.axis_index("c")
    s = lax.axis_index("s")
    wid = s * _NC + c
    bufs = (rows_a, rows_b, rows_c, rows_d)
    gsems = (gsem_a, gsem_b, gsem_c, gsem_d)

    pltpu.sync_copy(g_hbm.at[pl.ds(s * _RPS, _RPS)],
                    acc_sh.at[pl.ds(s * _RPS, _RPS)])

    pltpu.sync_copy(row_hbm.at[wid], row_v)
    pltpu.sync_copy(col_hbm.at[wid], col_v)
    plsc.subcore_barrier()

    for b in range(_NBUF):
        pltpu.async_copy(g_hbm.at[row_v.at[b]], bufs[b], gsems[b])

    def _block(i, carry):
        j0 = i * _NBUF
        for b in range(_NBUF):
            j = j0 + b
            pltpu.make_async_copy(g_hbm.at[row_v.at[j]], bufs[b],
                                  gsems[b]).wait()
            pltpu.sync_copy(bufs[b], acc_sh.at[col_v.at[j]], add=True)

            @pl.when(j + _NBUF < _NCHK)
            def _(b=b, j=j):
                pltpu.async_copy(g_hbm.at[row_v.at[j + _NBUF]], bufs[b],
                                 gsems[b])
        return carry

    lax.fori_loop(0, _NCHK // _NBUF, _block, 0)
    plsc.subcore_barrier()

    pltpu.sync_copy(acc_sh.at[pl.ds(s * _RPS, _RPS)],
                    out_hbm.at[c, pl.ds(s * _RPS, _RPS)])


_prop = functools.partial(
    pl.kernel,
    out_type=jax.ShapeDtypeStruct((_NC, _N, _D_HID), jnp.float32),
    scratch_types=(
        [pltpu.VMEM((_NCHK, _CH), jnp.int32)] * 2
        + [pltpu.VMEM((_CH, _D_HID), jnp.float32)] * _NBUF
        + [pltpu.VMEM_SHARED((_NPAD, _D_HID), jnp.float32)]
        + [pltpu.SemaphoreType.DMA] * _NBUF
    ),
    mesh=_MESH,
    compiler_params=_SC_PARAMS,
)(_prop_body)


def _deg_body(ones_hbm, col_hbm, out_hbm, col_v, ones_v, acc_sh, sem):
    c = lax.axis_index("c")
    s = lax.axis_index("s")
    wid = s * _NC + c

    @pl.when(s == 0)
    def _():
        pltpu.sync_copy(ones_hbm, acc_sh.at[pl.ds(0, _N)])

    for k in range(_CH // 16):
        ones_v[pl.ds(k * 16, 16)] = jnp.ones((16,), jnp.float32)
    pltpu.sync_copy(col_hbm.at[wid], col_v)
    plsc.subcore_barrier()

    def _chunk(j, carry):
        pltpu.sync_copy(ones_v, acc_sh.at[col_v.at[j]], add=True)
        return carry

    lax.fori_loop(0, _NCHK, _chunk, 0)
    plsc.subcore_barrier()

    @pl.when(s == 0)
    def _():
        pltpu.sync_copy(acc_sh.at[pl.ds(0, _N)], out_hbm.at[c])


_deg = functools.partial(
    pl.kernel,
    out_type=jax.ShapeDtypeStruct((_NC, _N), jnp.float32),
    scratch_types=[
        pltpu.VMEM((_NCHK, _CH), jnp.int32),
        pltpu.VMEM((_CH,), jnp.float32),
        pltpu.VMEM_SHARED((_NPAD,), jnp.float32),
        pltpu.SemaphoreType.DMA,
    ],
    mesh=_MESH,
    compiler_params=_SC_PARAMS,
)(_deg_body)


def _lin1_body(x_ref, w_ref, dp0_ref, dp1_ref, g_ref, dv_ref):
    dv = lax.rsqrt(dp0_ref[...] + dp1_ref[...] - 1.0)
    h = jnp.dot(x_ref[...], w_ref[...], preferred_element_type=jnp.float32)
    g_ref[...] = h * dv
    dv_ref[...] = dv


def _mid_body(p0_ref, p1_ref, g1_ref, dv_ref, b1_ref, g2_ref):
    s = dv_ref[...] * (p0_ref[...] + p1_ref[...] - g1_ref[...])
    g2_ref[...] = dv_ref[...] * jnp.maximum(s + b1_ref[...], 0.0)


def _fin_body(q0_ref, q1_ref, g2_ref, dv_ref, w2_ref, b2_ref, out_ref):
    s = dv_ref[...] * (q0_ref[...] + q1_ref[...] - g2_ref[...])
    out_ref[...] = (
        jnp.dot(s, w2_ref[...], preferred_element_type=jnp.float32)
        + b2_ref[...]
    )


def _row_blk(d):
    return pl.BlockSpec((_BLK, d), lambda i: (i, 0))


def _full(shape):
    return pl.BlockSpec(shape, lambda i: (0, 0))


_lin1 = pl.pallas_call(
    _lin1_body,
    grid=(_NBLK,),
    in_specs=[_row_blk(_D_IN), _full((_D_IN, _D_HID)),
              _row_blk(1), _row_blk(1)],
    out_specs=[_row_blk(_D_HID), _row_blk(1)],
    out_shape=[jax.ShapeDtypeStruct((_N, _D_HID), jnp.float32),
               jax.ShapeDtypeStruct((_N, 1), jnp.float32)],
)

_mid = pl.pallas_call(
    _mid_body,
    grid=(_NBLK,),
    in_specs=[_row_blk(_D_HID), _row_blk(_D_HID), _row_blk(_D_HID),
              _row_blk(1), _full((1, _D_HID))],
    out_specs=_row_blk(_D_HID),
    out_shape=jax.ShapeDtypeStruct((_N, _D_HID), jnp.float32),
)

_fin = pl.pallas_call(
    _fin_body,
    grid=(_NBLK,),
    in_specs=[_row_blk(_D_HID), _row_blk(_D_HID), _row_blk(_D_HID),
              _row_blk(1), _full((_D_HID, _D_OUT)), _full((1, _D_OUT))],
    out_specs=_row_blk(_D_OUT),
    out_shape=jax.ShapeDtypeStruct((_N, _D_OUT), jnp.float32),
)


def kernel(x, edge_index, W1, b1, W2, b2):
    row = edge_index[0].astype(jnp.int32).reshape(_NW, _E // _NW)
    col = edge_index[1].astype(jnp.int32).reshape(_NW, _E // _NW)
    npad = _EPW - _E // _NW
    rowp = jnp.concatenate(
        [row, jnp.zeros((_NW, npad), jnp.int32)],
        axis=1).reshape(_NW, _NCHK, _CH)
    padcol = jnp.broadcast_to(
        _N + (jnp.arange(npad, dtype=jnp.int32) % _NDIS), (_NW, npad))
    colp = jnp.concatenate([col, padcol], axis=1).reshape(_NW, _NCHK, _CH)

    ones_n = jnp.ones((_N,), dtype=jnp.float32)
    dp = _deg(ones_n, colp)
    dp0 = dp[0].reshape(_N, 1)
    dp1 = dp[1].reshape(_N, 1)

    g1, dv = _lin1(x, W1, dp0, dp1)

    p = _prop(g1, rowp, colp)
    g2 = _mid(p[0], p[1], g1, dv, b1.reshape(1, _D_HID))

    q = _prop(g2, rowp, colp)
    out = _fin(q[0], q[1], g2, dv, W2, b2.reshape(1, _D_OUT))
    return out


# speedup vs baseline: 1.0681x; 1.0681x over previous
import functools

import jax
import jax.numpy as jnp
from jax import lax
from jax.experimental import pallas as pl
from jax.experimental.pallas import tpu as pltpu
from jax.experimental.pallas import tpu_sc as plsc

_N = 10000
_E = 320000
_D_IN = 128
_D_HID = 16
_D_OUT = 40

_NC = 2
_NS = 16
_NW = _NC * _NS
_CH = 512
_NCHK = 20
_EPW = _CH * _NCHK
_EPAD = _EPW * _NW
_RPS = _N // _NS
_NDIS = 128
_NPAD = _N + _NDIS
_NBUF = 4

_BLK = 1000
_NBLK = _N // _BLK

_SC_PARAMS = pltpu.CompilerParams(use_tc_tiling_on_sc=False)
_MESH = plsc.VectorSubcoreMesh(core_axis_name="c", subcore_axis_name="s")


def _prop_body(g_hbm, row_hbm, col_hbm, out_hbm,
               row_v, col_v,
               rows_a, rows_b, rows_c, rows_d,
               acc_sh,
               gsem_a, gsem_b, gsem_c, gsem_d):
    c = lax.axis_index("c")
    s = lax.axis_index("s")
    wid = s * _NC + c
    bufs = (rows_a, rows_b, rows_c, rows_d)
    gsems = (gsem_a, gsem_b, gsem_c, gsem_d)

    pltpu.sync_copy(g_hbm.at[pl.ds(s * _RPS, _RPS)],
                    acc_sh.at[pl.ds(s * _RPS, _RPS)])

    pltpu.sync_copy(row_hbm.at[wid], row_v)
    pltpu.sync_copy(col_hbm.at[wid], col_v)
    plsc.subcore_barrier()

    for b in range(_NBUF):
        pltpu.async_copy(g_hbm.at[row_v.at[b]], bufs[b], gsems[b])

    def _block(i, carry):
        j0 = i * _NBUF
        for b in range(_NBUF):
            j = j0 + b
            pltpu.make_async_copy(g_hbm.at[row_v.at[j]], bufs[b],
                                  gsems[b]).wait()
            pltpu.sync_copy(bufs[b], acc_sh.at[col_v.at[j]], add=True)

            @pl.when(j + _NBUF < _NCHK)
            def _(b=b, j=j):
                pltpu.async_copy(g_hbm.at[row_v.at[j + _NBUF]], bufs[b],
                                 gsems[b])
        return carry

    lax.fori_loop(0, _NCHK // _NBUF, _block, 0)
    plsc.subcore_barrier()

    pltpu.sync_copy(acc_sh.at[pl.ds(s * _RPS, _RPS)],
                    out_hbm.at[c, pl.ds(s * _RPS, _RPS)])


_prop = functools.partial(
    pl.kernel,
    out_type=jax.ShapeDtypeStruct((_NC, _N, _D_HID), jnp.float32),
    scratch_types=(
        [pltpu.VMEM((_NCHK, _CH), jnp.int32)] * 2
        + [pltpu.VMEM((_CH, _D_HID), jnp.float32)] * _NBUF
        + [pltpu.VMEM_SHARED((_NPAD, _D_HID), jnp.float32)]
        + [pltpu.SemaphoreType.DMA] * _NBUF
    ),
    mesh=_MESH,
    compiler_params=_SC_PARAMS,
)(_prop_body)


def _deg_body(ones_hbm, col_hbm, out_hbm, col_v, ones_v, acc_sh, sem):
    c = lax.axis_index("c")
    s = lax.axis_index("s")
    wid = s * _NC + c

    @pl.when(s == 0)
    def _():
        pltpu.sync_copy(ones_hbm, acc_sh.at[pl.ds(0, _N)])

    for k in range(_CH // 16):
        ones_v[pl.ds(k * 16, 16)] = jnp.ones((16,), jnp.float32)
    pltpu.sync_copy(col_hbm.at[wid], col_v)
    plsc.subcore_barrier()

    def _chunk(j, carry):
        pltpu.sync_copy(ones_v, acc_sh.at[col_v.at[j]], add=True)
        return carry

    lax.fori_loop(0, _NCHK, _chunk, 0)
    plsc.subcore_barrier()

    @pl.when(s == 0)
    def _():
        pltpu.sync_copy(acc_sh.at[pl.ds(0, _N)], out_hbm.at[c])


_deg = functools.partial(
    pl.kernel,
    out_type=jax.ShapeDtypeStruct((_NC, _N), jnp.float32),
    scratch_types=[
        pltpu.VMEM((_NCHK, _CH), jnp.int32),
        pltpu.VMEM((_CH,), jnp.float32),
        pltpu.VMEM_SHARED((_NPAD,), jnp.float32),
        pltpu.SemaphoreType.DMA,
    ],
    mesh=_MESH,
    compiler_params=_SC_PARAMS,
)(_deg_body)


def _lin1_body(x_ref, w_ref, dp0_ref, dp1_ref, g_ref, dv_ref):
    dv = lax.rsqrt(dp0_ref[...] + dp1_ref[...] - 1.0)
    h = jnp.dot(x_ref[...], w_ref[...], preferred_element_type=jnp.float32)
    g_ref[...] = h * dv
    dv_ref[...] = dv


def _mid_body(p0_ref, p1_ref, g1_ref, dv_ref, b1_ref, g2_ref):
    s = dv_ref[...] * (p0_ref[...] + p1_ref[...] - g1_ref[...])
    g2_ref[...] = dv_ref[...] * jnp.maximum(s + b1_ref[...], 0.0)


def _fin_body(q0_ref, q1_ref, g2_ref, dv_ref, w2_ref, b2_ref, out_ref):
    s = dv_ref[...] * (q0_ref[...] + q1_ref[...] - g2_ref[...])
    out_ref[...] = (
        jnp.dot(s, w2_ref[...], preferred_element_type=jnp.float32)
        + b2_ref[...]
    )


def _row_blk(d):
    return pl.BlockSpec((_BLK, d), lambda i: (i, 0))


def _full(shape):
    return pl.BlockSpec(shape, lambda i: (0, 0))


_lin1 = pl.pallas_call(
    _lin1_body,
    grid=(_NBLK,),
    in_specs=[_row_blk(_D_IN), _full((_D_IN, _D_HID)),
              _row_blk(1), _row_blk(1)],
    out_specs=[_row_blk(_D_HID), _row_blk(1)],
    out_shape=[jax.ShapeDtypeStruct((_N, _D_HID), jnp.float32),
               jax.ShapeDtypeStruct((_N, 1), jnp.float32)],
)

_mid = pl.pallas_call(
    _mid_body,
    grid=(_NBLK,),
    in_specs=[_row_blk(_D_HID), _row_blk(_D_HID), _row_blk(_D_HID),
              _row_blk(1), _full((1, _D_HID))],
    out_specs=_row_blk(_D_HID),
    out_shape=jax.ShapeDtypeStruct((_N, _D_HID), jnp.float32),
)

_fin = pl.pallas_call(
    _fin_body,
    grid=(_NBLK,),
    in_specs=[_row_blk(_D_HID), _row_blk(_D_HID), _row_blk(_D_HID),
              _row_blk(1), _full((_D_HID, _D_OUT)), _full((1, _D_OUT))],
    out_specs=_row_blk(_D_OUT),
    out_shape=jax.ShapeDtypeStruct((_N, _D_OUT), jnp.float32),
)


def kernel(x, edge_index, W1, b1, W2, b2):
    row = edge_index[0].astype(jnp.int32).reshape(_NW, _E // _NW)
    col = edge_index[1].astype(jnp.int32).reshape(_NW, _E // _NW)
    npad = _EPW - _E // _NW
    rowp = jnp.concatenate(
        [row, jnp.zeros((_NW, npad), jnp.int32)],
        axis=1).reshape(_NW, _NCHK, _CH)
    padcol = jnp.broadcast_to(
        _N + (jnp.arange(npad, dtype=jnp.int32) % _NDIS), (_NW, npad))
    colp = jnp.concatenate([col, padcol], axis=1).reshape(_NW, _NCHK, _CH)

    ones_n = jnp.ones((_N,), dtype=jnp.float32)
    dp = _deg(ones_n, colp)
    dp0 = dp[0].reshape(_N, 1)
    dp1 = dp[1].reshape(_N, 1)

    g1, dv = _lin1(x, W1, dp0, dp1)

    p = _prop(g1, rowp, colp)
    g2 = _mid(p[0], p[1], g1, dv, b1.reshape(1, _D_HID))

    q = _prop(g2, rowp, colp)
    out = _fin(q[0], q[1], g2, dv, W2, b2.reshape(1, _D_OUT))
    return out


# speedup vs baseline: 1.0932x; 1.0235x over previous
import functools

import jax
import jax.numpy as jnp
from jax import lax
from jax.experimental import pallas as pl
from jax.experimental.pallas import tpu as pltpu
from jax.experimental.pallas import tpu_sc as plsc

_N = 10000
_E = 320000
_D_IN = 128
_D_HID = 16
_D_OUT = 40

_NC = 2
_NS = 16
_NW = _NC * _NS
_CH = 512
_NCHK = 20
_EPW = _CH * _NCHK
_EPAD = _EPW * _NW
_RPS = _N // _NS
_NDIS = 128
_NPAD = _N + _NDIS
_NBUF = 4

_BLK = 1000
_NBLK = _N // _BLK

_SC_PARAMS = pltpu.CompilerParams(use_tc_tiling_on_sc=False)
_MESH = plsc.VectorSubcoreMesh(core_axis_name="c", subcore_axis_name="s")


def _prop_body(g_hbm, row_hbm, col_hbm, out_hbm,
               row_v, col_v,
               rows_a, rows_b, rows_c, rows_d,
               acc_sh,
               gsem_a, gsem_b, gsem_c, gsem_d):
    c = lax.axis_index("c")
    s = lax.axis_index("s")
    wid = s * _NC + c
    bufs = (rows_a, rows_b, rows_c, rows_d)
    gsems = (gsem_a, gsem_b, gsem_c, gsem_d)

    pltpu.sync_copy(g_hbm.at[pl.ds(s * _RPS, _RPS)],
                    acc_sh.at[pl.ds(s * _RPS, _RPS)])

    pltpu.sync_copy(row_hbm.at[wid], row_v)
    pltpu.sync_copy(col_hbm.at[wid], col_v)
    plsc.subcore_barrier()

    for b in range(_NBUF):
        pltpu.async_copy(g_hbm.at[row_v.at[b]], bufs[b], gsems[b])

    def _block(i, carry):
        j0 = i * _NBUF
        for b in range(_NBUF):
            j = j0 + b
            pltpu.make_async_copy(g_hbm.at[row_v.at[j]], bufs[b],
                                  gsems[b]).wait()
            pltpu.sync_copy(bufs[b], acc_sh.at[col_v.at[j]], add=True)

            @pl.when(j + _NBUF < _NCHK)
            def _(b=b, j=j):
                pltpu.async_copy(g_hbm.at[row_v.at[j + _NBUF]], bufs[b],
                                 gsems[b])
        return carry

    lax.fori_loop(0, _NCHK // _NBUF, _block, 0)
    plsc.subcore_barrier()

    pltpu.sync_copy(acc_sh.at[pl.ds(s * _RPS, _RPS)],
                    out_hbm.at[c, pl.ds(s * _RPS, _RPS)])


_prop = functools.partial(
    pl.kernel,
    out_type=jax.ShapeDtypeStruct((_NC, _N, _D_HID), jnp.float32),
    scratch_types=(
        [pltpu.VMEM((_NCHK, _CH), jnp.int32)] * 2
        + [pltpu.VMEM((_CH, _D_HID), jnp.float32)] * _NBUF
        + [pltpu.VMEM_SHARED((_NPAD, _D_HID), jnp.float32)]
        + [pltpu.SemaphoreType.DMA] * _NBUF
    ),
    mesh=_MESH,
    compiler_params=_SC_PARAMS,
)(_prop_body)


def _deg_body(ones_hbm, col_hbm, out_hbm, col_v, ones_v, acc_sh, sem):
    c = lax.axis_index("c")
    s = lax.axis_index("s")
    wid = s * _NC + c

    @pl.when(s == 0)
    def _():
        pltpu.sync_copy(ones_hbm, acc_sh.at[pl.ds(0, _N)])

    for k in range(_CH // 16):
        ones_v[pl.ds(k * 16, 16)] = jnp.ones((16,), jnp.float32)
    pltpu.sync_copy(col_hbm.at[wid], col_v)
    plsc.subcore_barrier()

    def _chunk(j, carry):
        pltpu.sync_copy(ones_v, acc_sh.at[col_v.at[j]], add=True)
        return carry

    lax.fori_loop(0, _NCHK, _chunk, 0)
    plsc.subcore_barrier()

    @pl.when(s == 0)
    def _():
        pltpu.sync_copy(acc_sh.at[pl.ds(0, _N)], out_hbm.at[c])


_deg = functools.partial(
    pl.kernel,
    out_type=jax.ShapeDtypeStruct((_NC, _N), jnp.float32),
    scratch_types=[
        pltpu.VMEM((_NCHK, _CH), jnp.int32),
        pltpu.VMEM((_CH,), jnp.float32),
        pltpu.VMEM_SHARED((_NPAD,), jnp.float32),
        pltpu.SemaphoreType.DMA,
    ],
    mesh=_MESH,
    compiler_params=_SC_PARAMS,
)(_deg_body)


def _mm_body(x_ref, w_ref, h_ref):
    h_ref[...] = jnp.dot(x_ref[...], w_ref[...],
                         preferred_element_type=jnp.float32)


def _scale_body(h_ref, dp0_ref, dp1_ref, g_ref, dv_ref):
    dv = lax.rsqrt(dp0_ref[...] + dp1_ref[...] - 1.0)
    g_ref[...] = h_ref[...] * dv
    dv_ref[...] = dv


def _mid_body(p0_ref, p1_ref, g1_ref, dv_ref, b1_ref, g2_ref):
    s = dv_ref[...] * (p0_ref[...] + p1_ref[...] - g1_ref[...])
    g2_ref[...] = dv_ref[...] * jnp.maximum(s + b1_ref[...], 0.0)


def _fin_body(q0_ref, q1_ref, g2_ref, dv_ref, w2_ref, b2_ref, out_ref):
    s = dv_ref[...] * (q0_ref[...] + q1_ref[...] - g2_ref[...])
    out_ref[...] = (
        jnp.dot(s, w2_ref[...], preferred_element_type=jnp.float32)
        + b2_ref[...]
    )


_mm = pl.pallas_call(
    _mm_body,
    out_shape=jax.ShapeDtypeStruct((_N, _D_HID), jnp.float32),
)

_scale = pl.pallas_call(
    _scale_body,
    out_shape=[jax.ShapeDtypeStruct((_N, _D_HID), jnp.float32),
               jax.ShapeDtypeStruct((_N, 1), jnp.float32)],
)

_mid = pl.pallas_call(
    _mid_body,
    out_shape=jax.ShapeDtypeStruct((_N, _D_HID), jnp.float32),
)

_fin = pl.pallas_call(
    _fin_body,
    out_shape=jax.ShapeDtypeStruct((_N, _D_OUT), jnp.float32),
)


def kernel(x, edge_index, W1, b1, W2, b2):
    row = edge_index[0].astype(jnp.int32).reshape(_NW, _E // _NW)
    col = edge_index[1].astype(jnp.int32).reshape(_NW, _E // _NW)
    npad = _EPW - _E // _NW
    rowp = jnp.concatenate(
        [row, jnp.zeros((_NW, npad), jnp.int32)],
        axis=1).reshape(_NW, _NCHK, _CH)
    padcol = jnp.broadcast_to(
        _N + (jnp.arange(npad, dtype=jnp.int32) % _NDIS), (_NW, npad))
    colp = jnp.concatenate([col, padcol], axis=1).reshape(_NW, _NCHK, _CH)

    ones_n = jnp.ones((_N,), dtype=jnp.float32)
    h = _mm(x, W1)
    dp = _deg(ones_n, colp)
    dp0 = dp[0].reshape(_N, 1)
    dp1 = dp[1].reshape(_N, 1)

    g1, dv = _scale(h, dp0, dp1)

    p = _prop(g1, rowp, colp)
    g2 = _mid(p[0], p[1], g1, dv, b1.reshape(1, _D_HID))

    q = _prop(g2, rowp, colp)
    out = _fin(q[0], q[1], g2, dv, W2, b2.reshape(1, _D_OUT))
    return out


# speedup vs baseline: 1.1787x; 1.0782x over previous
import functools

import jax
import jax.numpy as jnp
from jax import lax
from jax.experimental import pallas as pl
from jax.experimental.pallas import tpu as pltpu
from jax.experimental.pallas import tpu_sc as plsc

_N = 10000
_E = 320000
_D_IN = 128
_D_HID = 16
_D_OUT = 40

_NC = 2
_NS = 16
_NW = _NC * _NS
_CH = 512
_NCHK = 20
_EPW = _CH * _NCHK
_EPAD = _EPW * _NW
_RPS = _N // _NS
_NDIS = 128
_NPAD = _N + _NDIS
_NBUF = 4

_BLK = 1000
_NBLK = _N // _BLK

_SC_PARAMS = pltpu.CompilerParams(use_tc_tiling_on_sc=False)
_MESH = plsc.VectorSubcoreMesh(core_axis_name="c", subcore_axis_name="s")


def _prop_body(g_hbm, row_hbm, col_hbm, out0_hbm, out1_hbm,
               row_v, col_v,
               rows_a, rows_b, rows_c, rows_d,
               acc_sh,
               gsem_a, gsem_b, gsem_c, gsem_d):
    c = lax.axis_index("c")
    s = lax.axis_index("s")
    wid = s * _NC + c
    bufs = (rows_a, rows_b, rows_c, rows_d)
    gsems = (gsem_a, gsem_b, gsem_c, gsem_d)

    pltpu.sync_copy(g_hbm.at[pl.ds(s * _RPS, _RPS)],
                    acc_sh.at[pl.ds(s * _RPS, _RPS)])

    pltpu.sync_copy(row_hbm.at[wid], row_v)
    pltpu.sync_copy(col_hbm.at[wid], col_v)
    plsc.subcore_barrier()

    for b in range(_NBUF):
        pltpu.async_copy(g_hbm.at[row_v.at[b]], bufs[b], gsems[b])

    def _block(i, carry):
        j0 = i * _NBUF
        for b in range(_NBUF):
            j = j0 + b
            pltpu.make_async_copy(g_hbm.at[row_v.at[j]], bufs[b],
                                  gsems[b]).wait()
            pltpu.sync_copy(bufs[b], acc_sh.at[col_v.at[j]], add=True)

            @pl.when(j + _NBUF < _NCHK)
            def _(b=b, j=j):
                pltpu.async_copy(g_hbm.at[row_v.at[j + _NBUF]], bufs[b],
                                 gsems[b])
        return carry

    lax.fori_loop(0, _NCHK // _NBUF, _block, 0)
    plsc.subcore_barrier()

    @pl.when(c == 0)
    def _():
        pltpu.sync_copy(acc_sh.at[pl.ds(s * _RPS, _RPS)],
                        out0_hbm.at[pl.ds(s * _RPS, _RPS)])

    @pl.when(c == 1)
    def _():
        pltpu.sync_copy(acc_sh.at[pl.ds(s * _RPS, _RPS)],
                        out1_hbm.at[pl.ds(s * _RPS, _RPS)])


_prop = functools.partial(
    pl.kernel,
    out_type=[jax.ShapeDtypeStruct((_N, _D_HID), jnp.float32),
              jax.ShapeDtypeStruct((_N, _D_HID), jnp.float32)],
    scratch_types=(
        [pltpu.VMEM((_NCHK, _CH), jnp.int32)] * 2
        + [pltpu.VMEM((_CH, _D_HID), jnp.float32)] * _NBUF
        + [pltpu.VMEM_SHARED((_NPAD, _D_HID), jnp.float32)]
        + [pltpu.SemaphoreType.DMA] * _NBUF
    ),
    mesh=_MESH,
    compiler_params=_SC_PARAMS,
)(_prop_body)


def _deg_body(ones_hbm, col_hbm, out_hbm, col_v, ones_v, acc_sh, sem):
    c = lax.axis_index("c")
    s = lax.axis_index("s")
    wid = s * _NC + c

    @pl.when(s == 0)
    def _():
        pltpu.sync_copy(ones_hbm, acc_sh.at[pl.ds(0, _N)])

    for k in range(_CH // 16):
        ones_v[pl.ds(k * 16, 16)] = jnp.ones((16,), jnp.float32)
    pltpu.sync_copy(col_hbm.at[wid], col_v)
    plsc.subcore_barrier()

    def _chunk(j, carry):
        pltpu.sync_copy(ones_v, acc_sh.at[col_v.at[j]], add=True)
        return carry

    lax.fori_loop(0, _NCHK, _chunk, 0)
    plsc.subcore_barrier()

    @pl.when(s == 0)
    def _():
        pltpu.sync_copy(acc_sh.at[pl.ds(0, _N)], out_hbm.at[c])


_deg = functools.partial(
    pl.kernel,
    out_type=jax.ShapeDtypeStruct((_NC, _N), jnp.float32),
    scratch_types=[
        pltpu.VMEM((_NCHK, _CH), jnp.int32),
        pltpu.VMEM((_CH,), jnp.float32),
        pltpu.VMEM_SHARED((_NPAD,), jnp.float32),
        pltpu.SemaphoreType.DMA,
    ],
    mesh=_MESH,
    compiler_params=_SC_PARAMS,
)(_deg_body)


_NV = _N // 8


def _mm_body(x8_ref, w1b_ref, h_ref):
    h_ref[...] = jnp.dot(x8_ref[...], w1b_ref[...],
                         preferred_element_type=jnp.float32)


def _scale_body(h_ref, dp_ref, g_ref, dv_ref):
    dv = lax.rsqrt(dp_ref[0] + dp_ref[1] - 1.0)
    g_ref[...] = h_ref[...] * dv
    dv_ref[...] = dv


def _mid_body(p0_ref, p1_ref, g1_ref, dv_ref, b1_ref, g2_ref):
    s = dv_ref[...] * (p0_ref[...] + p1_ref[...] - g1_ref[...])
    g2_ref[...] = dv_ref[...] * jnp.maximum(s + b1_ref[...], 0.0)


def _fin_body(q0_ref, q1_ref, g2_ref, dv_ref, w2b_ref, b2_ref, out_ref):
    s = dv_ref[...] * (q0_ref[...] + q1_ref[...] - g2_ref[...])
    out_ref[...] = (
        jnp.dot(s, w2b_ref[...], preferred_element_type=jnp.float32)
        + b2_ref[...]
    )


_mm = pl.pallas_call(
    _mm_body,
    out_shape=jax.ShapeDtypeStruct((_NV, 128), jnp.float32),
)

_scale = pl.pallas_call(
    _scale_body,
    out_shape=[jax.ShapeDtypeStruct((_NV, 128), jnp.float32),
               jax.ShapeDtypeStruct((_NV, 128), jnp.float32)],
)

_mid = pl.pallas_call(
    _mid_body,
    out_shape=jax.ShapeDtypeStruct((_NV, 128), jnp.float32),
)

_fin = pl.pallas_call(
    _fin_body,
    out_shape=jax.ShapeDtypeStruct((_NV, 8 * _D_OUT), jnp.float32),
)


def _bdiag(w):
    return jax.scipy.linalg.block_diag(*([w] * 8))


def kernel(x, edge_index, W1, b1, W2, b2):
    row = edge_index[0].astype(jnp.int32)
    col = edge_index[1].astype(jnp.int32)
    npad = _EPAD - _E
    rowp = jnp.concatenate(
        [row, jnp.zeros((npad,), jnp.int32)]).reshape(_NW, _NCHK, _CH)
    padcol = _N + (jnp.arange(npad, dtype=jnp.int32) % _NDIS)
    colp = jnp.concatenate([col, padcol]).reshape(_NW, _NCHK, _CH)

    ones_n = jnp.ones((_N,), dtype=jnp.float32)
    h = _mm(x.reshape(_NV, 8 * _D_IN), _bdiag(W1))
    dp = _deg(ones_n, colp)
    dp_rep = jnp.repeat(dp[:, :, None], _D_HID, axis=2).reshape(2, _NV, 128)

    g1, dv = _scale(h, dp_rep)

    p0, p1 = _prop(g1.reshape(_N, _D_HID), rowp, colp)
    g2 = _mid(p0.reshape(_NV, 128), p1.reshape(_NV, 128), g1, dv,
              jnp.tile(b1, 8).reshape(1, 128))

    q0, q1 = _prop(g2.reshape(_N, _D_HID), rowp, colp)
    out8 = _fin(q0.reshape(_NV, 128), q1.reshape(_NV, 128), g2, dv,
                _bdiag(W2), jnp.tile(b2, 8).reshape(1, 8 * _D_OUT))
    return out8.reshape(_N, _D_OUT)


# speedup vs baseline: 2.3155x; 1.9645x over previous
import functools

import jax
import jax.numpy as jnp
from jax import lax
from jax.experimental import pallas as pl
from jax.experimental.pallas import tpu as pltpu
from jax.experimental.pallas import tpu_sc as plsc

_N = 10000
_E = 320000
_D_IN = 128
_D_HID = 16
_D_OUT = 40

_NC = 2
_NS = 16
_NW = _NC * _NS
_EPW = _E // _NW
_CH = 512
_NFULL = _EPW // _CH
_TAIL = _EPW - _NFULL * _CH
_RPS = _N // _NS
_NBUF = 4

_NV = _N // 8

_SC_PARAMS = pltpu.CompilerParams(use_tc_tiling_on_sc=False)
_MESH = plsc.VectorSubcoreMesh(core_axis_name="c", subcore_axis_name="s")


def _prop_body(g_hbm, ei_hbm, out0_hbm, out1_hbm,
               row_v, col_v,
               rows_a, rows_b, rows_c, rows_d,
               acc_sh,
               gsem_a, gsem_b, gsem_c, gsem_d):
    c = lax.axis_index("c")
    s = lax.axis_index("s")
    wid = s * _NC + c
    bufs = (rows_a, rows_b, rows_c, rows_d)
    gsems = (gsem_a, gsem_b, gsem_c, gsem_d)

    pltpu.sync_copy(g_hbm.at[pl.ds(s * _RPS, _RPS)],
                    acc_sh.at[pl.ds(s * _RPS, _RPS)])

    base = wid * _EPW
    pltpu.sync_copy(ei_hbm.at[0, pl.ds(base, _EPW)], row_v)
    pltpu.sync_copy(ei_hbm.at[1, pl.ds(base, _EPW)], col_v)
    plsc.subcore_barrier()

    def _gather(j, buf, sem, n):
        return pltpu.async_copy(g_hbm.at[row_v.at[pl.ds(j * _CH, n)]],
                                buf, sem)

    def _wait(j, buf, sem, n):
        pltpu.make_async_copy(g_hbm.at[row_v.at[pl.ds(j * _CH, n)]],
                              buf, sem).wait()

    def _scatter(j, buf, n):
        pltpu.sync_copy(buf, acc_sh.at[col_v.at[pl.ds(j * _CH, n)]],
                        add=True)

    for b in range(_NBUF):
        _gather(b, bufs[b], gsems[b], _CH)

    def _block(i, carry):
        j0 = i * _NBUF
        for b in range(_NBUF):
            j = j0 + b
            _wait(j, bufs[b], gsems[b], _CH)
            _scatter(j, bufs[b], _CH)

            @pl.when(j + _NBUF < _NFULL)
            def _(b=b, j=j):
                _gather(j + _NBUF, bufs[b], gsems[b], _CH)
        return carry

    lax.fori_loop(0, _NFULL // _NBUF, _block, 0)

    for j in range(_NFULL - _NFULL % _NBUF, _NFULL):
        b = j % _NBUF
        _wait(j, bufs[b], gsems[b], _CH)
        _scatter(j, bufs[b], _CH)

    tb = bufs[(_NFULL % _NBUF)]
    tail = tb.at[pl.ds(0, _TAIL)]
    _gather(_NFULL, tail, gsems[0], _TAIL)
    _wait(_NFULL, tail, gsems[0], _TAIL)
    _scatter(_NFULL, tail, _TAIL)
    plsc.subcore_barrier()

    @pl.when(c == 0)
    def _():
        pltpu.sync_copy(acc_sh.at[pl.ds(s * _RPS, _RPS)],
                        out0_hbm.at[pl.ds(s * _RPS, _RPS)])

    @pl.when(c == 1)
    def _():
        pltpu.sync_copy(acc_sh.at[pl.ds(s * _RPS, _RPS)],
                        out1_hbm.at[pl.ds(s * _RPS, _RPS)])


_prop = functools.partial(
    pl.kernel,
    out_type=[jax.ShapeDtypeStruct((_N, _D_HID), jnp.float32),
              jax.ShapeDtypeStruct((_N, _D_HID), jnp.float32)],
    scratch_types=(
        [pltpu.VMEM((_EPW,), jnp.int32)] * 2
        + [pltpu.VMEM((_CH, _D_HID), jnp.float32)] * _NBUF
        + [pltpu.VMEM_SHARED((_N, _D_HID), jnp.float32)]
        + [pltpu.SemaphoreType.DMA] * _NBUF
    ),
    mesh=_MESH,
    compiler_params=_SC_PARAMS,
)(_prop_body)


def _deg_body(ones_hbm, ei_hbm, out_hbm, col_v, ones_v, acc_sh, sem):
    c = lax.axis_index("c")
    s = lax.axis_index("s")
    wid = s * _NC + c

    @pl.when(s == 0)
    def _():
        pltpu.sync_copy(ones_hbm, acc_sh)

    for k in range(_CH // 16):
        ones_v[pl.ds(k * 16, 16)] = jnp.ones((16,), jnp.float32)
    pltpu.sync_copy(ei_hbm.at[1, pl.ds(wid * _EPW, _EPW)], col_v)
    plsc.subcore_barrier()

    def _chunk(j, carry):
        pltpu.sync_copy(ones_v, acc_sh.at[col_v.at[pl.ds(j * _CH, _CH)]],
                        add=True)
        return carry

    lax.fori_loop(0, _NFULL, _chunk, 0)
    pltpu.sync_copy(ones_v.at[pl.ds(0, _TAIL)],
                    acc_sh.at[col_v.at[pl.ds(_NFULL * _CH, _TAIL)]],
                    add=True)
    plsc.subcore_barrier()

    @pl.when(s == 0)
    def _():
        pltpu.sync_copy(acc_sh, out_hbm.at[c])


_deg = functools.partial(
    pl.kernel,
    out_type=jax.ShapeDtypeStruct((_NC, _N), jnp.float32),
    scratch_types=[
        pltpu.VMEM((_EPW,), jnp.int32),
        pltpu.VMEM((_CH,), jnp.float32),
        pltpu.VMEM_SHARED((_N,), jnp.float32),
        pltpu.SemaphoreType.DMA,
    ],
    mesh=_MESH,
    compiler_params=_SC_PARAMS,
)(_deg_body)


def _mm_body(x8_ref, w1b_ref, h_ref):
    h_ref[...] = jnp.dot(x8_ref[...], w1b_ref[...],
                         preferred_element_type=jnp.float32)


def _scale_body(h_ref, dp_ref, g_ref, dv_ref):
    dv = lax.rsqrt(dp_ref[0] + dp_ref[1] - 1.0)
    g_ref[...] = h_ref[...] * dv
    dv_ref[...] = dv


def _mid_body(p0_ref, p1_ref, g1_ref, dv_ref, b1_ref, g2_ref):
    s = dv_ref[...] * (p0_ref[...] + p1_ref[...] - g1_ref[...])
    g2_ref[...] = dv_ref[...] * jnp.maximum(s + b1_ref[...], 0.0)


def _fin_body(q0_ref, q1_ref, g2_ref, dv_ref, w2b_ref, b2_ref, out_ref):
    s = dv_ref[...] * (q0_ref[...] + q1_ref[...] - g2_ref[...])
    out_ref[...] = (
        jnp.dot(s, w2b_ref[...], preferred_element_type=jnp.float32)
        + b2_ref[...]
    )


_mm = pl.pallas_call(
    _mm_body,
    out_shape=jax.ShapeDtypeStruct((_NV, 128), jnp.float32),
)

_scale = pl.pallas_call(
    _scale_body,
    out_shape=[jax.ShapeDtypeStruct((_NV, 128), jnp.float32),
               jax.ShapeDtypeStruct((_NV, 128), jnp.float32)],
)

_mid = pl.pallas_call(
    _mid_body,
    out_shape=jax.ShapeDtypeStruct((_NV, 128), jnp.float32),
)

_fin = pl.pallas_call(
    _fin_body,
    out_shape=jax.ShapeDtypeStruct((_NV, 8 * _D_OUT), jnp.float32),
)


def _bdiag(w):
    return jax.scipy.linalg.block_diag(*([w] * 8))


def _view(t):
    return t.reshape(_NV, 128)


def kernel(x, edge_index, W1, b1, W2, b2):
    ei = edge_index.astype(jnp.int32)
    ones_n = jnp.ones((_N,), dtype=jnp.float32)

    h = _mm(x.reshape(_NV, 8 * _D_IN), _bdiag(W1))
    dp = _deg(ones_n, ei)
    dp_rep = jnp.repeat(dp[:, :, None], _D_HID, axis=2).reshape(2, _NV, 128)

    g1, dv = _scale(h, dp_rep)

    p0, p1 = _prop(g1.reshape(_N, _D_HID), ei)
    g2 = _mid(_view(p0), _view(p1), g1, dv, jnp.tile(b1, 8).reshape(1, 128))

    q0, q1 = _prop(g2.reshape(_N, _D_HID), ei)
    out8 = _fin(_view(q0), _view(q1), g2, dv,
                _bdiag(W2), jnp.tile(b2, 8).reshape(1, 8 * _D_OUT))
    return out8.reshape(_N, _D_OUT)


# speedup vs baseline: 2.6366x; 1.1387x over previous
import functools

import jax
import jax.numpy as jnp
from jax import lax
from jax.experimental import pallas as pl
from jax.experimental.pallas import tpu as pltpu
from jax.experimental.pallas import tpu_sc as plsc

_N = 10000
_E = 320000
_D_IN = 128
_D_HID = 16
_D_OUT = 40

_NC = 2
_NS = 16
_NW = _NC * _NS
_EPW = _E // _NW
_CH = 512
_NFULL = _EPW // _CH
_TAIL = _EPW - _NFULL * _CH
_RPS = _N // _NS
_NBUF = 4

_NV = _N // 8

_SC_PARAMS = pltpu.CompilerParams(use_tc_tiling_on_sc=False)
_MESH = plsc.VectorSubcoreMesh(core_axis_name="c", subcore_axis_name="s")


def _prop_body(g_hbm, ei_hbm, out0_hbm, out1_hbm,
               row_v, col_v,
               rows_a, rows_b, rows_c, rows_d,
               acc_sh,
               gsem_a, gsem_b, gsem_c, gsem_d):
    c = lax.axis_index("c")
    s = lax.axis_index("s")
    wid = s * _NC + c
    bufs = (rows_a, rows_b, rows_c, rows_d)
    gsems = (gsem_a, gsem_b, gsem_c, gsem_d)

    pltpu.sync_copy(g_hbm.at[pl.ds(s * _RPS, _RPS)],
                    acc_sh.at[pl.ds(s * _RPS, _RPS)])

    base = wid * _EPW
    pltpu.sync_copy(ei_hbm.at[0, pl.ds(base, _EPW)], row_v)
    pltpu.sync_copy(ei_hbm.at[1, pl.ds(base, _EPW)], col_v)
    plsc.subcore_barrier()

    def _gather(j, buf, sem, n):
        return pltpu.async_copy(g_hbm.at[row_v.at[pl.ds(j * _CH, n)]],
                                buf, sem)

    def _wait(j, buf, sem, n):
        pltpu.make_async_copy(g_hbm.at[row_v.at[pl.ds(j * _CH, n)]],
                              buf, sem).wait()

    def _scatter(j, buf, n):
        pltpu.sync_copy(buf, acc_sh.at[col_v.at[pl.ds(j * _CH, n)]],
                        add=True)

    for b in range(_NBUF):
        _gather(b, bufs[b], gsems[b], _CH)

    def _block(i, carry):
        j0 = i * _NBUF
        for b in range(_NBUF):
            j = j0 + b
            _wait(j, bufs[b], gsems[b], _CH)
            _scatter(j, bufs[b], _CH)

            @pl.when(j + _NBUF < _NFULL)
            def _(b=b, j=j):
                _gather(j + _NBUF, bufs[b], gsems[b], _CH)
        return carry

    lax.fori_loop(0, _NFULL // _NBUF, _block, 0)

    for j in range(_NFULL - _NFULL % _NBUF, _NFULL):
        b = j % _NBUF
        _wait(j, bufs[b], gsems[b], _CH)
        _scatter(j, bufs[b], _CH)

    tb = bufs[(_NFULL % _NBUF)]
    tail = tb.at[pl.ds(0, _TAIL)]
    _gather(_NFULL, tail, gsems[0], _TAIL)
    _wait(_NFULL, tail, gsems[0], _TAIL)
    _scatter(_NFULL, tail, _TAIL)
    plsc.subcore_barrier()

    @pl.when(c == 0)
    def _():
        pltpu.sync_copy(acc_sh.at[pl.ds(s * _RPS, _RPS)],
                        out0_hbm.at[pl.ds(s * _RPS, _RPS)])

    @pl.when(c == 1)
    def _():
        pltpu.sync_copy(acc_sh.at[pl.ds(s * _RPS, _RPS)],
                        out1_hbm.at[pl.ds(s * _RPS, _RPS)])


_prop = functools.partial(
    pl.kernel,
    out_type=[jax.ShapeDtypeStruct((_N, _D_HID), jnp.float32),
              jax.ShapeDtypeStruct((_N, _D_HID), jnp.float32)],
    scratch_types=(
        [pltpu.VMEM((_EPW,), jnp.int32)] * 2
        + [pltpu.VMEM((_CH, _D_HID), jnp.float32)] * _NBUF
        + [pltpu.VMEM_SHARED((_N, _D_HID), jnp.float32)]
        + [pltpu.SemaphoreType.DMA] * _NBUF
    ),
    mesh=_MESH,
    compiler_params=_SC_PARAMS,
)(_prop_body)


def _deg_body(ones_hbm, ei_hbm, out_hbm, col_v, ones_v, acc_sh, sem):
    c = lax.axis_index("c")
    s = lax.axis_index("s")
    wid = s * _NC + c

    @pl.when(s == 0)
    def _():
        pltpu.sync_copy(ones_hbm, acc_sh)

    for k in range(_CH // 16):
        ones_v[pl.ds(k * 16, 16)] = jnp.ones((16,), jnp.float32)
    pltpu.sync_copy(ei_hbm.at[1, pl.ds(wid * _EPW, _EPW)], col_v)
    plsc.subcore_barrier()

    def _chunk(j, carry):
        pltpu.sync_copy(ones_v, acc_sh.at[col_v.at[pl.ds(j * _CH, _CH)]],
                        add=True)
        return carry

    lax.fori_loop(0, _NFULL, _chunk, 0)
    pltpu.sync_copy(ones_v.at[pl.ds(0, _TAIL)],
                    acc_sh.at[col_v.at[pl.ds(_NFULL * _CH, _TAIL)]],
                    add=True)
    plsc.subcore_barrier()

    @pl.when(s == 0)
    def _():
        pltpu.sync_copy(acc_sh, out_hbm.at[c])


_deg = functools.partial(
    pl.kernel,
    out_type=jax.ShapeDtypeStruct((_NC, _N), jnp.float32),
    scratch_types=[
        pltpu.VMEM((_EPW,), jnp.int32),
        pltpu.VMEM((_CH,), jnp.float32),
        pltpu.VMEM_SHARED((_N,), jnp.float32),
        pltpu.SemaphoreType.DMA,
    ],
    mesh=_MESH,
    compiler_params=_SC_PARAMS,
)(_deg_body)


def _mm_body(x8_ref, w1b_ref, h_ref):
    h_ref[...] = jnp.dot(x8_ref[...], w1b_ref[...],
                         preferred_element_type=jnp.float32)


def _scale_body(h_ref, dp0_ref, dp1_ref, g_ref, dv_ref):
    dv = lax.rsqrt(dp0_ref[...] + dp1_ref[...] - 1.0)
    g_ref[...] = h_ref[...] * dv
    dv_ref[...] = dv


def _mid_body(p0_ref, p1_ref, g1_ref, dv_ref, b1_ref, g2_ref):
    s = dv_ref[...] * (p0_ref[...] + p1_ref[...] - g1_ref[...])
    g2_ref[...] = dv_ref[...] * jnp.maximum(s + b1_ref[...], 0.0)


def _fin_body(q0_ref, q1_ref, g2_ref, dv_ref, w2b_ref, b2_ref, out_ref):
    s = dv_ref[...] * (q0_ref[...] + q1_ref[...] - g2_ref[...])
    out_ref[...] = (
        jnp.dot(s, w2b_ref[...], preferred_element_type=jnp.float32)
        + b2_ref[...]
    )


_mm = pl.pallas_call(
    _mm_body,
    out_shape=jax.ShapeDtypeStruct((_NV, 128), jnp.float32),
)

_scale = pl.pallas_call(
    _scale_body,
    out_shape=[jax.ShapeDtypeStruct((_NV, 128), jnp.float32),
               jax.ShapeDtypeStruct((_NV, 128), jnp.float32)],
)

_mid = pl.pallas_call(
    _mid_body,
    out_shape=jax.ShapeDtypeStruct((_NV, 128), jnp.float32),
)

_fin = pl.pallas_call(
    _fin_body,
    out_shape=jax.ShapeDtypeStruct((_NV, 8 * _D_OUT), jnp.float32),
)


def _bdiag(w):
    return jax.scipy.linalg.block_diag(*([w] * 8))


def _view(t):
    return t.reshape(_NV, 128)


def kernel(x, edge_index, W1, b1, W2, b2):
    ei = edge_index.astype(jnp.int32)
    ones_n = jnp.ones((_N,), dtype=jnp.float32)

    h = _mm(x.reshape(_NV, 8 * _D_IN), _bdiag(W1))
    dp = _deg(ones_n, ei)
    def _rep(v):
        return jnp.broadcast_to(
            v.reshape(_NV, 8)[:, :, None], (_NV, 8, _D_HID)
        ).reshape(_NV, 128)

    g1, dv = _scale(h, _rep(dp[0]), _rep(dp[1]))

    p0, p1 = _prop(g1.reshape(_N, _D_HID), ei)
    g2 = _mid(_view(p0), _view(p1), g1, dv, jnp.tile(b1, 8).reshape(1, 128))

    q0, q1 = _prop(g2.reshape(_N, _D_HID), ei)
    out8 = _fin(_view(q0), _view(q1), g2, dv,
                _bdiag(W2), jnp.tile(b2, 8).reshape(1, 8 * _D_OUT))
    return out8.reshape(_N, _D_OUT)
